# Initial kernel scaffold; baseline (speedup 1.0000x reference)
import jax
import jax.numpy as jnp
from jax.experimental import pallas as pl


def kernel(coords, feats, params):
    raise NotImplementedError("write your pallas kernel here")


# speedup vs baseline: 16.4002x; 16.4002x over previous
import functools

import jax
import jax.numpy as jnp
import numpy as np
from jax import lax
from jax.experimental import pallas as pl
from jax.experimental.pallas import tpu as pltpu
from jax.experimental.pallas import tpu_sc as plsc

_K = 16
_SLOPE = 0.2
_EPS = 1e-5
_NT = (((1,), (1,)), ((), ()))
_NN = (((1,), (0,)), ((), ()))


def _xx_row_kernel(xt_ref, o_ref):
    x2 = xt_ref[...] * xt_ref[...]
    ones = jnp.ones((1, x2.shape[1]), jnp.float32)
    o_ref[...] = lax.dot_general(ones, x2, _NT,
                                 preferred_element_type=jnp.float32,
                                 precision=lax.Precision.HIGHEST)


def _knn_kernel(xt_blk_ref, xt_full_ref, xxr_ref, idx_ref, *, R, N):
    xb = xt_blk_ref[...]
    inner2 = 2.0 * lax.dot_general(xb.astype(jnp.bfloat16),
                                   xt_full_ref[...].astype(jnp.bfloat16), _NT,
                                   preferred_element_type=jnp.float32)
    xxc = jnp.sum(xb * xb, axis=1, keepdims=True)
    d = inner2 - xxc - xxr_ref[...]
    iota = lax.broadcasted_iota(jnp.int32, (R, N), 1)
    cols = []
    for _ in range(_K):
        m = jnp.max(d, axis=1, keepdims=True)
        sel = jnp.where(d >= m, iota, N)
        j = jnp.min(sel, axis=1, keepdims=True)
        cols.append(j)
        d = jnp.where(iota == j, -jnp.inf, d)
    idx_ref[...] = jnp.concatenate(cols, axis=1)


def _knn(xt, R=256):
    N, C = xt.shape
    xxr = pl.pallas_call(
        _xx_row_kernel,
        grid=(N // 512,),
        in_specs=[pl.BlockSpec((512, C), lambda i: (i, 0))],
        out_specs=pl.BlockSpec((1, 512), lambda i: (0, i)),
        out_shape=jax.ShapeDtypeStruct((1, N), jnp.float32),
    )(xt)
    kern = functools.partial(_knn_kernel, R=R, N=N)
    return pl.pallas_call(
        kern,
        grid=(N // R,),
        in_specs=[
            pl.BlockSpec((R, C), lambda i: (i, 0)),
            pl.BlockSpec((N, C), lambda i: (0, 0)),
            pl.BlockSpec((1, N), lambda i: (0, 0)),
        ],
        out_specs=pl.BlockSpec((R, _K), lambda i: (i, 0)),
        out_shape=jax.ShapeDtypeStruct((N, _K), jnp.int32),
    )(xt, xt, xxr)


def _sc_gather(xt, idx):
    N, C = xt.shape
    M = idx.shape[0]
    info = plsc.get_sparse_core_info()
    NW = info.num_cores * info.num_subcores
    m_per_w = M // NW
    CH = 128
    n_ch = m_per_w // CH
    mesh = plsc.VectorSubcoreMesh(core_axis_name="c", subcore_axis_name="s")

    @functools.partial(
        pl.kernel, mesh=mesh,
        compiler_params=pltpu.CompilerParams(use_tc_tiling_on_sc=False),
        out_type=jax.ShapeDtypeStruct((M, C), jnp.float32),
        scratch_types=[
            pltpu.VMEM((CH,), jnp.int32),
            pltpu.VMEM((CH, C), jnp.float32),
            pltpu.SemaphoreType.DMA,
        ],
    )
    def k(table_hbm, idx_hbm, out_hbm, idx_v, rows_v, sem):
        wid = lax.axis_index("s") * info.num_cores + lax.axis_index("c")
        base = wid * m_per_w
        def body(g, carry):
            off = base + g * CH
            pltpu.sync_copy(idx_hbm.at[pl.ds(off, CH)], idx_v)
            pltpu.async_copy(table_hbm.at[idx_v], rows_v, sem).wait()
            pltpu.sync_copy(rows_v, out_hbm.at[pl.ds(off, CH)])
            return carry
        lax.fori_loop(0, n_ch, body, 0)

    return k(xt, idx)


def _edge_kernel(x_ref, nbr_ref, w1_ref, w2_ref, s_ref, be_ref, b_ref,
                 o_ref, *, R):
    C = x_ref.shape[1]
    Co = w1_ref.shape[1]
    xb = x_ref[...]
    nbr = nbr_ref[...]
    xrep = jnp.broadcast_to(xb[:, None, :], (R, _K, C)).reshape(R * _K, C)
    diff = (nbr - xrep).astype(jnp.bfloat16)
    e = lax.dot_general(diff, w1_ref[...].astype(jnp.bfloat16), _NN,
                        preferred_element_type=jnp.float32)
    u = lax.dot_general(xb.astype(jnp.bfloat16),
                        w2_ref[...].astype(jnp.bfloat16), _NN,
                        preferred_element_type=jnp.float32)
    u = u + b_ref[...]
    s = s_ref[...].reshape(1, 1, Co)
    be = be_ref[...].reshape(1, 1, Co)
    pre = (e.reshape(R, _K, Co) + u[:, None, :]) * s + be
    act = jnp.where(pre >= 0, pre, _SLOPE * pre)
    o_ref[...] = jnp.max(act, axis=1)


def _edge_block(xt, cp, bp, c_log):
    N, Cpad = xt.shape
    W = cp["w"]
    Co = W.shape[0]
    pad = Cpad - c_log
    zpad = jnp.zeros((pad, Co), jnp.float32) if pad else None
    W1 = W[:, :c_log].T
    W2 = W[:, c_log:].T
    if pad:
        W1 = jnp.concatenate([W1, zpad], axis=0)
        W2 = jnp.concatenate([W2, zpad], axis=0)
    idx = _knn(xt)
    nbr = _sc_gather(xt, idx.reshape(N * _K))
    s = (bp["g"] / np.sqrt(1.0 + _EPS)).astype(jnp.float32)
    R = 256
    kern = functools.partial(_edge_kernel, R=R)
    out = pl.pallas_call(
        kern,
        grid=(N // R,),
        in_specs=[
            pl.BlockSpec((R, Cpad), lambda i: (i, 0)),
            pl.BlockSpec((R * _K, Cpad), lambda i: (i, 0)),
            pl.BlockSpec((Cpad, Co), lambda i: (0, 0)),
            pl.BlockSpec((Cpad, Co), lambda i: (0, 0)),
            pl.BlockSpec((1, Co), lambda i: (0, 0)),
            pl.BlockSpec((1, Co), lambda i: (0, 0)),
            pl.BlockSpec((1, Co), lambda i: (0, 0)),
        ],
        out_specs=pl.BlockSpec((R, Co), lambda i: (i, 0)),
        out_shape=jax.ShapeDtypeStruct((N, Co), jnp.float32),
    )(xt, nbr, W1, W2, s.reshape(1, -1), bp["be"].reshape(1, -1),
      cp["b"].reshape(1, -1))
    return out


def _mm_kernel(x_ref, a_ref, s_ref, t_ref, o_ref, *, act):
    r = lax.dot_general(x_ref[...].astype(jnp.bfloat16),
                        a_ref[...].astype(jnp.bfloat16), _NN,
                        preferred_element_type=jnp.float32)
    r = r * s_ref[...] + t_ref[...]
    if act:
        r = jnp.where(r >= 0, r, _SLOPE * r)
    o_ref[...] = r


def _mm(xt, A, s, t, act, Nb=512):
    N, Cin = xt.shape
    Cout = A.shape[1]
    kern = functools.partial(_mm_kernel, act=act)
    return pl.pallas_call(
        kern,
        grid=(N // Nb,),
        in_specs=[
            pl.BlockSpec((Nb, Cin), lambda i: (i, 0)),
            pl.BlockSpec((Cin, Cout), lambda i: (0, 0)),
            pl.BlockSpec((1, Cout), lambda i: (0, 0)),
            pl.BlockSpec((1, Cout), lambda i: (0, 0)),
        ],
        out_specs=pl.BlockSpec((Nb, Cout), lambda i: (i, 0)),
        out_shape=jax.ShapeDtypeStruct((N, Cout), jnp.float32),
    )(xt, A, s.reshape(1, -1), t.reshape(1, -1))


def _forward_one(xt0, params):
    x1 = _edge_block(xt0, params["ec1"], params["bn1"], 4)
    x2 = _edge_block(x1, params["ec2"], params["bn2"], 64)
    x3 = _edge_block(x2, params["ec3"], params["bn3"], 64)
    x4 = _edge_block(x3, params["ec4"], params["bn4"], 128)
    xc = jnp.concatenate([x1, x2, x3, x4], axis=1)
    s5 = (params["bn5"]["g"] / np.sqrt(1.0 + _EPS)).astype(jnp.float32)
    t5 = params["c5"]["b"] * s5 + params["bn5"]["be"]
    x5 = _mm(xc, params["c5"]["w"].T, s5, t5, act=True)
    Af = jnp.concatenate([fp["w"].T for fp in params["feat"]], axis=1)
    sf = jnp.concatenate([(bp["g"] / np.sqrt(1.0 + _EPS)).astype(jnp.float32)
                          for bp in params["obn"]])
    tf = jnp.concatenate([fp["b"] * (bp["g"] / np.sqrt(1.0 + _EPS)).astype(jnp.float32)
                          + bp["be"]
                          for fp, bp in zip(params["feat"], params["obn"])])
    ms = _mm(x5, Af, sf, tf, act=False)
    ms0, ms1, ms2 = ms[:, :256], ms[:, 256:512], ms[:, 512:]
    sem = _mm(ms2, params["sem"]["w"].T,
              jnp.ones((20,), jnp.float32), params["sem"]["b"], act=False)
    return ms0, ms1, ms2, sem


def kernel(coords, feats, params):
    B, N, _ = coords.shape
    x0 = jnp.concatenate(
        [coords, feats[:, :, 3:4], jnp.zeros((B, N, 12), jnp.float32)],
        axis=-1)
    outs = [_forward_one(x0[b], params) for b in range(B)]
    ms0 = jnp.stack([o[0] for o in outs])
    ms1 = jnp.stack([o[1] for o in outs])
    ms2 = jnp.stack([o[2] for o in outs])
    sem = jnp.stack([o[3] for o in outs])
    masks = jnp.zeros((B, N), dtype=bool)
    return (ms0, ms1, ms2, coords, coords, coords, masks, masks, masks, sem)
